# Initial kernel scaffold; baseline (speedup 1.0000x reference)
import jax
import jax.numpy as jnp
from jax.experimental import pallas as pl


def kernel(node_features_set, edge_index, edge_features_set, target_node_set, params):
    raise NotImplementedError("write your pallas kernel here")


# speedup vs baseline: 341.5860x; 341.5860x over previous
import functools

import jax
import jax.numpy as jnp
from jax.experimental import pallas as pl
from jax.experimental.pallas import tpu as pltpu

B, N, E = 2, 24, 256
I = 256
P = 256
K = N * N

_INTERPRET = False


def _dg(a, b, ca, cb):
    return jax.lax.dot_general(a, b, (((ca,), (cb,)), ((), ())),
                               precision=jax.lax.Precision.HIGHEST,
                               preferred_element_type=jnp.float32)


def _iota(shape, dim):
    return jax.lax.broadcasted_iota(jnp.int32, shape, dim).astype(jnp.float32)


def _tr(x, eye):
    return _dg(x, eye, 0, 0)


def _k0_body(nf_ref, ei_ref, ef_ref, tn_ref,
             m0w, m0b, m1w, m1b, m2w, m2b,
             g_ref, gt_ref, m0n_ref, mt0_ref):
    nf = nf_ref[0]
    src = ei_ref[0, 0:1, :]
    dst = ei_ref[0, 1:2, :]
    srcf = src.astype(jnp.float32)
    dstf = dst.astype(jnp.float32)
    oh_src = (_iota((N, E), 0) == srcf).astype(jnp.float32)
    oh_dst = (_iota((N, E), 0) == dstf).astype(jnp.float32)
    nfsrc = _dg(oh_src, nf, 0, 0)
    nfdst = _dg(oh_dst, nf, 0, 0)
    seq = jnp.concatenate([nfsrc, ef_ref[0], nfdst], axis=1)
    x = _dg(seq, m0w[...], 1, 1) + m0b[...]
    x = _dg(x, m1w[...], 1, 1) + m1b[...]
    x = _dg(x, m2w[...], 1, 1) + m2b[...]

    g = _dg(nf, nf, 1, 1)
    g_ref[0] = g
    gt = _dg(nf, tn_ref[0], 1, 1)
    gt_ref[0] = gt
    gtdst = _dg(oh_dst, gt, 0, 0)
    mt0_ref[0] = _dg(oh_src, x * gtdst, 1, 0)
    gd = _dg(oh_dst, g, 0, 0)
    eye24 = (_iota((N, N), 0) == _iota((N, N), 1)).astype(jnp.float32)
    oh_src_ev = _tr(oh_src, eye24)
    for v in range(N):
        w_col = oh_src_ev[:, v:v + 1]
        blk = _dg(gd, x * w_col, 0, 0)
        m0n_ref[0, v * N:(v + 1) * N, :] = blk


def _k1_body(ei_ref, w0_ref, w1_ref, meta_ref):
    src = ei_ref[0, 0:1, :].astype(jnp.float32)
    dst = ei_ref[0, 1:2, :].astype(jnp.float32)
    eye24 = (_iota((N, N), 0) == _iota((N, N), 1)).astype(jnp.float32)
    eyeK = (_iota((K, K), 0) == _iota((K, K), 1)).astype(jnp.float32)
    oh_src = (_iota((N, E), 0) == src).astype(jnp.float32)
    oh_dst = (_iota((N, E), 0) == dst).astype(jnp.float32)
    a_cnt = _dg(oh_src, oh_dst, 1, 1)
    af = (a_cnt > 0).astype(jnp.float32)
    atf = _tr(af, eye24)

    iota_e = _iota((N, E), 1)
    firsts = jnp.min(jnp.where(oh_src > 0, iota_e, float(E)), axis=1,
                     keepdims=True)
    firsts_r = _tr(firsts, eye24)
    rank = jnp.sum((firsts_r < firsts).astype(jnp.float32), axis=1,
                   keepdims=True)
    rank_r = _tr(rank, eye24)
    iota_s = _iota((N, N), 0)
    iota_l = _iota((N, N), 1)
    order_r = jnp.sum(jnp.where(rank == iota_l, iota_s, 0.0), axis=0,
                      keepdims=True)

    fcols = []
    for d in range(N):
        m = oh_src * (dst == float(d)).astype(jnp.float32)
        fcols.append(jnp.min(jnp.where(m > 0, iota_e, float(E)), axis=1,
                             keepdims=True))
    F = jnp.concatenate(fcols, axis=1)
    rcols = []
    for j in range(N):
        fj = F[:, j:j + 1]
        lt = (F < fj) | ((F == fj) & (iota_l < float(j)))
        rcols.append(jnp.sum(lt.astype(jnp.float32), axis=1, keepdims=True))
    nbr_rank = jnp.concatenate(rcols, axis=1)
    nbr_rank_t = _tr(nbr_rank, eye24)

    at3 = jnp.broadcast_to(atf[None, :, :], (N, N, N))
    nbrk3 = jnp.broadcast_to(nbr_rank_t[None, :, :], (N, N, N))
    iota_u3 = _iota((N, N, N), 2)

    def bfs_round(_, st):
        visited, pos, parent, frontier, tail = st
        f3 = jnp.broadcast_to(frontier[:, None, :], (N, N, N))
        mask3 = f3 * at3
        pos3 = jnp.broadcast_to(pos[:, None, :], (N, N, N))
        minpos = jnp.min(jnp.where(mask3 > 0, pos3, float(N)), axis=2)
        newv = (minpos < float(N)).astype(jnp.float32) * (1.0 - visited)
        par_new = jnp.min(
            jnp.where((mask3 > 0) & (pos3 == minpos[:, :, None]), iota_u3,
                      float(N)), axis=2)
        par_c = jnp.minimum(par_new, float(N - 1))
        nbrk = jnp.sum(
            jnp.where(iota_u3 == par_c[:, :, None], nbrk3, 0.0), axis=2)
        key = jnp.where(newv > 0, minpos * 32.0 + nbrk, 1e6)
        r = jnp.sum((key[:, None, :] < key[:, :, None]).astype(jnp.float32),
                    axis=2)
        pos = jnp.where(newv > 0, tail + r, pos)
        parent = jnp.where(newv > 0, par_c, parent)
        tail = tail + jnp.sum(newv, axis=1, keepdims=True)
        visited = jnp.maximum(visited, newv)
        return (visited, pos, parent, newv, tail)

    visited0 = eye24
    pos0 = jnp.where(eye24 > 0, 0.0, float(N))
    parent0 = jnp.zeros((N, N), jnp.float32)
    tail0 = jnp.ones((N, 1), jnp.float32)
    visited, pos, parent, _, _ = jax.lax.fori_loop(
        0, N, bfs_round, (visited0, pos0, parent0, visited0, tail0))

    k_col = _iota((K, 1), 0)
    kdiv = jnp.floor(k_col / float(N))
    nn_col = k_col - kdiv * float(N)
    iota_kl24 = _iota((K, N), 1)
    oh_kdiv = (iota_kl24 == kdiv).astype(jnp.float32)
    node_col = jnp.sum(oh_kdiv * order_r, axis=1, keepdims=True)
    oh_node = (iota_kl24 == node_col).astype(jnp.float32)
    oh_nn = (iota_kl24 == nn_col).astype(jnp.float32)

    arow_node = _dg(oh_node, af, 1, 0)
    neighbor = jnp.sum(arow_node * oh_nn, axis=1, keepdims=True)
    vrow = _dg(oh_node, visited, 1, 0)
    prow = _dg(oh_node, pos, 1, 0)
    parrow = _dg(oh_node, parent, 1, 0)
    acol_nn = _dg(oh_nn, atf, 1, 0)
    valid = vrow * acol_nn
    exists = ((jnp.sum(valid, axis=1, keepdims=True) > 0)
              & (node_col != nn_col)).astype(jnp.float32)
    minp = jnp.min(jnp.where(valid > 0, prow, float(N)), axis=1, keepdims=True)
    u_col = jnp.min(jnp.where((valid > 0) & (prow == minp), iota_kl24,
                              float(N)), axis=1, keepdims=True)
    u_col = jnp.minimum(u_col, float(N - 1))

    chs = [u_col]
    for t in range(1, N):
        prev = chs[t - 1]
        oh_prev = (iota_kl24 == prev).astype(jnp.float32)
        pg = jnp.sum(oh_prev * parrow, axis=1, keepdims=True)
        chs.append(jnp.where(prev == node_col, node_col, pg))
    ch = jnp.concatenate(chs, axis=1)
    depth = jnp.min(jnp.where(ch == node_col, iota_kl24, float(N)), axis=1,
                    keepdims=True)
    depth = jnp.minimum(depth, float(N - 1))

    paths = []
    for j in range(N + 1):
        gi = jnp.clip(depth - float(j), 0.0, float(N - 1))
        oh_gi = (iota_kl24 == gi).astype(jnp.float32)
        pj = jnp.sum(oh_gi * ch, axis=1, keepdims=True)
        paths.append(jnp.where(float(j) == depth + 1.0, nn_col, pj))

    iota_klK = _iota((K, K), 1)
    w0 = jnp.zeros((K, K), jnp.float32)
    w1 = jnp.zeros((K, K), jnp.float32)
    for t in range(N):
        pa, pb = paths[t], paths[t + 1]
        tm = (float(t) <= depth).astype(jnp.float32)
        w0 = w0 + tm * (iota_klK == pa * float(N) + pb).astype(jnp.float32)
        w1 = w1 + tm * (iota_klK == pb * float(N) + pa).astype(jnp.float32)
    w0_ref[0] = w0
    w1_ref[0] = w1

    arow_nn = _dg(oh_nn, af, 1, 0)
    arev = jnp.sum(arow_nn * oh_node, axis=1, keepdims=True)
    pidx_nn = jnp.sum(oh_nn * rank_r, axis=1, keepdims=True)
    cov = (arev == 0) & (pidx_nn * float(N) + node_col < k_col)
    do = exists * (1.0 - neighbor) * (1.0 - cov.astype(jnp.float32))

    do_r = _tr(do, eyeK)
    iota_ksub = _iota((K, K), 0)
    iota_klK2 = _iota((K, K), 1)
    rkdo = jnp.sum(do_r * (iota_klK2 < iota_ksub).astype(jnp.float32), axis=1,
                   keepdims=True)
    rkdo_r = _tr(rkdo, eyeK)
    node_r = _tr(node_col, eyeK)
    nn_r = _tr(nn_col, eyeK)
    sel = do_r * (rkdo_r == iota_ksub).astype(jnp.float32)
    pid_c = jnp.sum(sel * iota_klK2, axis=1, keepdims=True)
    node_c = jnp.sum(sel * node_r, axis=1, keepdims=True)
    nn_c = jnp.sum(sel * nn_r, axis=1, keepdims=True)
    cnt = jnp.sum(do_r, axis=1, keepdims=True)
    meta_ref[0, 0:1, :] = _tr(pid_c, eyeK).astype(jnp.int32)
    meta_ref[0, 1:2, :] = _tr(node_c, eyeK).astype(jnp.int32)
    meta_ref[0, 2:3, :] = _tr(nn_c, eyeK).astype(jnp.int32)
    meta_ref[0, 3:4, :] = jnp.broadcast_to(cnt, (1, K)).astype(jnp.int32)


def _k2_body(pidc, nodec, nnc, countc,
             w0_ref, w1_ref, m0n_ref, mt0_ref, g_ref, gt_ref,
             nf_ref, tn_ref,
             f0w, f0b, f1w, f1b, f2w, f2b, lnw, lnb,
             i0w, i0b, i1w, i1b, i2w, i2b,
             out_ref, mn_ref, mt_ref):
    b = pl.program_id(0)
    mn_ref[...] = m0n_ref[0]
    mt_ref[...] = mt0_ref[0]
    lw = lnw[...]
    lb = lnb[...]

    def body(j, carry):
        pid = pidc[b * K + j]
        node = nodec[b * K + j]
        nn = nnc[b * K + j]
        w0 = w0_ref[0, pl.ds(pid, 1), :]
        w1 = w1_ref[0, pl.ds(pid, 1), :]
        w = jnp.concatenate([w0, w1], axis=0)
        aggs = _dg(w, mn_ref[...], 1, 0)
        y = _dg(aggs, f0w[...], 1, 1) + f0b[...]
        y = _dg(y, f1w[...], 1, 1) + f1b[...]
        y = _dg(y, f2w[...], 1, 1) + f2b[...]
        z = y + aggs
        mu = jnp.mean(z, axis=1, keepdims=True)
        d = z - mu
        var = jnp.mean(d * d, axis=1, keepdims=True)
        zn = d * jax.lax.rsqrt(var + 1e-5) * lw + lb
        y0 = zn[0:1, :]
        y1 = zn[1:2, :]
        g_nn = g_ref[0, pl.ds(nn, 1), :]
        g_node = g_ref[0, pl.ds(node, 1), :]
        upd0 = _dg(g_nn, y0, 0, 0)
        upd1 = _dg(g_node, y1, 0, 0)
        mn_ref[pl.ds(node * N, N), :] = mn_ref[pl.ds(node * N, N), :] + upd0
        mn_ref[pl.ds(nn * N, N), :] = mn_ref[pl.ds(nn * N, N), :] + upd1
        gt_nn = gt_ref[0, pl.ds(nn, 1), :]
        gt_node = gt_ref[0, pl.ds(node, 1), :]
        mt_ref[pl.ds(node, 1), :] = mt_ref[pl.ds(node, 1), :] + gt_nn * y0
        mt_ref[pl.ds(nn, 1), :] = mt_ref[pl.ds(nn, 1), :] + gt_node * y1
        return carry

    jax.lax.fori_loop(0, countc[b], body, 0, unroll=False)

    mt = mt_ref[...]
    mu = jnp.mean(mt, axis=1, keepdims=True)
    d = mt - mu
    var = jnp.mean(d * d, axis=1, keepdims=True)
    retrieved = d * jax.lax.rsqrt(var + 1e-5) * lw + lb
    tgt = jnp.broadcast_to(tn_ref[0], (N, I))
    inp = jnp.concatenate([nf_ref[0], retrieved, tgt], axis=1)
    y = _dg(inp, i0w[...], 1, 1) + i0b[...]
    y = _dg(y, i1w[...], 1, 1) + i1b[...]
    y = _dg(y, i2w[...], 1, 1) + i2b[...]
    out_ref[0] = y


def _full(shape):
    return pl.BlockSpec(shape, lambda b: tuple(0 for _ in shape))


def _bat(shape):
    return pl.BlockSpec((1,) + shape, lambda b: (b,) + tuple(0 for _ in shape))


@jax.jit
def kernel(node_features_set, edge_index, edge_features_set, target_node_set,
           params):
    p = params
    nf = node_features_set
    ei = edge_index.astype(jnp.int32)
    ef = edge_features_set
    tn = target_node_set.reshape(B, 1, I)
    row = lambda v: v.reshape(1, -1)

    g, gt, m0n, mt0 = pl.pallas_call(
        _k0_body,
        grid=(B,),
        in_specs=[_bat((N, I)), _bat((2, E)), _bat((E, I)), _bat((1, I)),
                  _full((I, 3 * I)), _full((1, I)), _full((I, I)),
                  _full((1, I)), _full((P, I)), _full((1, P))],
        out_specs=[_bat((N, N)), _bat((N, 1)), _bat((K, P)), _bat((N, P))],
        out_shape=[jax.ShapeDtypeStruct((B, N, N), jnp.float32),
                   jax.ShapeDtypeStruct((B, N, 1), jnp.float32),
                   jax.ShapeDtypeStruct((B, K, P), jnp.float32),
                   jax.ShapeDtypeStruct((B, N, P), jnp.float32)],
        interpret=_INTERPRET,
    )(nf, ei, ef, tn,
      p['mlp0_W'], row(p['mlp0_b']), p['mlp1_W'], row(p['mlp1_b']),
      p['mlp2_W'], row(p['mlp2_b']))

    w0, w1, meta = pl.pallas_call(
        _k1_body,
        grid=(B,),
        in_specs=[_bat((2, E))],
        out_specs=[_bat((K, K)), _bat((K, K)), _bat((4, K))],
        out_shape=[jax.ShapeDtypeStruct((B, K, K), jnp.float32),
                   jax.ShapeDtypeStruct((B, K, K), jnp.float32),
                   jax.ShapeDtypeStruct((B, 4, K), jnp.int32)],
        interpret=_INTERPRET,
    )(ei)

    pidc = meta[:, 0, :].reshape(B * K)
    nodec = meta[:, 1, :].reshape(B * K)
    nnc = meta[:, 2, :].reshape(B * K)
    countc = meta[:, 3, 0].reshape(B)

    smem = pl.BlockSpec(memory_space=pltpu.SMEM)
    out = pl.pallas_call(
        _k2_body,
        grid=(B,),
        in_specs=[smem, smem, smem, smem,
                  _bat((K, K)), _bat((K, K)), _bat((K, P)), _bat((N, P)),
                  _bat((N, N)), _bat((N, 1)), _bat((N, I)), _bat((1, I)),
                  _full((P, P)), _full((1, P)), _full((P, P)), _full((1, P)),
                  _full((P, P)), _full((1, P)), _full((1, P)), _full((1, P)),
                  _full((I, 2 * I + P)), _full((1, I)), _full((I, I)),
                  _full((1, I)), _full((I, I)), _full((1, I))],
        out_specs=_bat((N, I)),
        out_shape=jax.ShapeDtypeStruct((B, N, I), jnp.float32),
        scratch_shapes=[pltpu.VMEM((K, P), jnp.float32),
                        pltpu.VMEM((N, P), jnp.float32)],
        interpret=_INTERPRET,
    )(pidc, nodec, nnc, countc,
      w0, w1, m0n, mt0, g, gt, nf, tn,
      p['fp0_W'], row(p['fp0_b']), p['fp1_W'], row(p['fp1_b']),
      p['fp2_W'], row(p['fp2_b']), row(p['ln_w']), row(p['ln_b']),
      p['inf0_W'], row(p['inf0_b']), p['inf1_W'], row(p['inf1_b']),
      p['inf2_W'], row(p['inf2_b']))
    return out
